# baseline (speedup 1.0000x reference)
import jax
import jax.numpy as jnp
import numpy as np
from jax.experimental import pallas as pl
from jax.experimental.pallas import tpu as pltpu

DIM_ = 512
CDIM_ = 13
NQ_ = 8
ROW_BLOCK = 1024


def _lfq_body(x_ref, win_ref, bin_ref, wout_ref, bout_ref, out_ref, idx_ref):
    x = x_ref[...]
    w_in = win_ref[...]
    h = jax.lax.dot_general(
        w_in, x, (((1,), (1,)), ((), ())),
        preferred_element_type=jnp.float32)
    h = h + bin_ref[...]

    pow2 = jax.lax.broadcasted_iota(jnp.int32, (CDIM_, 1), 0)
    pow2 = jnp.exp2(pow2.astype(jnp.float32))

    r = h
    q = jnp.zeros_like(h)
    idx_rows = []
    for i in range(NQ_):
        s = float(2.0 ** (-i))
        bits = r > 0
        hard = jnp.where(bits, s, -s)
        r = r - hard
        q = q + hard
        row = jnp.sum(jnp.where(bits, pow2, 0.0), axis=0, keepdims=True)
        idx_rows.append(row)
    idx_t = jnp.concatenate(idx_rows, axis=0)
    idx_ref[...] = idx_t.T.astype(jnp.int32)

    out = jax.lax.dot_general(
        q, wout_ref[...], (((0,), (1,)), ((), ())),
        preferred_element_type=jnp.float32)
    out_ref[...] = out + bout_ref[...]


def kernel(x, W_in, b_in, W_out, b_out):
    B, N, D = x.shape
    M = B * N
    T = ROW_BLOCK
    xm = x.reshape(M, D)
    bin2 = b_in.reshape(CDIM_, 1)
    bout2 = b_out.reshape(1, D)
    grid = (M // T,)
    out, idx = pl.pallas_call(
        _lfq_body,
        grid=grid,
        in_specs=[
            pl.BlockSpec((T, D), lambda g: (g, 0)),
            pl.BlockSpec((CDIM_, D), lambda g: (0, 0)),
            pl.BlockSpec((CDIM_, 1), lambda g: (0, 0)),
            pl.BlockSpec((D, CDIM_), lambda g: (0, 0)),
            pl.BlockSpec((1, D), lambda g: (0, 0)),
        ],
        out_specs=[
            pl.BlockSpec((T, D), lambda g: (g, 0)),
            pl.BlockSpec((T, NQ_), lambda g: (g, 0)),
        ],
        out_shape=[
            jax.ShapeDtypeStruct((M, D), jnp.float32),
            jax.ShapeDtypeStruct((M, NQ_), jnp.int32),
        ],
    )(xm, W_in, bin2, W_out, bout2)
    losses = jnp.zeros((NQ_,), x.dtype)
    return out.reshape(B, N, D), idx.reshape(B, N, NQ_), losses


# speedup vs baseline: 1.1215x; 1.1215x over previous
import jax
import jax.numpy as jnp
import numpy as np
from jax.experimental import pallas as pl
from jax.experimental.pallas import tpu as pltpu

DIM_ = 512
CDIM_ = 13
NQ_ = 8
ROW_BLOCK = 4096


def _lfq_body(x_ref, win_ref, bin_ref, wout_ref, bout_ref, out_ref, idx_ref):
    x = x_ref[...]
    w_in = win_ref[...]
    h = jax.lax.dot_general(
        w_in, x, (((1,), (1,)), ((), ())),
        preferred_element_type=jnp.float32)
    h = h + bin_ref[...]

    pow2 = jax.lax.broadcasted_iota(jnp.int32, (CDIM_, 1), 0)
    pow2 = jnp.exp2(pow2.astype(jnp.float32))

    r = h
    q = jnp.zeros_like(h)
    idx_rows = []
    for i in range(NQ_):
        s = float(2.0 ** (-i))
        bits = r > 0
        hard = jnp.where(bits, s, -s)
        r = r - hard
        q = q + hard
        row = jnp.sum(jnp.where(bits, pow2, 0.0), axis=0, keepdims=True)
        idx_rows.append(row)
    idx_t = jnp.concatenate(idx_rows, axis=0)
    idx_ref[...] = idx_t.T.astype(jnp.int32)

    out = jax.lax.dot_general(
        q, wout_ref[...], (((0,), (1,)), ((), ())),
        preferred_element_type=jnp.float32)
    out_ref[...] = out + bout_ref[...]


def kernel(x, W_in, b_in, W_out, b_out):
    B, N, D = x.shape
    M = B * N
    T = ROW_BLOCK
    xm = x.reshape(M, D)
    bin2 = b_in.reshape(CDIM_, 1)
    bout2 = b_out.reshape(1, D)
    grid = (M // T,)
    out, idx = pl.pallas_call(
        _lfq_body,
        grid=grid,
        compiler_params=pltpu.CompilerParams(
            dimension_semantics=("parallel",)),
        in_specs=[
            pl.BlockSpec((T, D), lambda g: (g, 0)),
            pl.BlockSpec((CDIM_, D), lambda g: (0, 0)),
            pl.BlockSpec((CDIM_, 1), lambda g: (0, 0)),
            pl.BlockSpec((D, CDIM_), lambda g: (0, 0)),
            pl.BlockSpec((1, D), lambda g: (0, 0)),
        ],
        out_specs=[
            pl.BlockSpec((T, D), lambda g: (g, 0)),
            pl.BlockSpec((T, NQ_), lambda g: (g, 0)),
        ],
        out_shape=[
            jax.ShapeDtypeStruct((M, D), jnp.float32),
            jax.ShapeDtypeStruct((M, NQ_), jnp.int32),
        ],
    )(xm, W_in, bin2, W_out, bout2)
    losses = jnp.zeros((NQ_,), x.dtype)
    return out.reshape(B, N, D), idx.reshape(B, N, NQ_), losses
